# baseline (speedup 1.0000x reference)
import jax
import jax.numpy as jnp
import numpy as np
from jax.experimental import pallas as pl
from jax.experimental.pallas import tpu as pltpu

ROWS = 128
COLS = 128
NCHUNK = 4
CROWS = ROWS // NCHUNK

THRESHOLD = np.float32(5e-8)
SCALE = np.float32(1e-7)


def _spl_loss_tc(x_hbm, out_ref, buf, sem):
    for i in range(NCHUNK):
        pltpu.make_async_copy(
            x_hbm.at[pl.ds(i * CROWS, CROWS)],
            buf.at[pl.ds(i * CROWS, CROWS)],
            sem.at[i],
        ).start()
    acc = None
    for i in range(NCHUNK):
        pltpu.make_async_copy(
            x_hbm.at[pl.ds(i * CROWS, CROWS)],
            buf.at[pl.ds(i * CROWS, CROWS)],
            sem.at[i],
        ).wait()
        x = buf[pl.ds(i * CROWS, CROWS), :]
        keep = (x * SCALE) < THRESHOLD
        y = jnp.where(keep, x, np.float32(0.0))
        p = jnp.sum(y, axis=0, keepdims=True)
        acc = p if acc is None else acc + p
    out_ref[...] = jnp.sum(acc, axis=1, keepdims=True)


def kernel(super_loss, index, v):
    del index, v
    x2d = super_loss.reshape(ROWS, COLS)
    out = pl.pallas_call(
        _spl_loss_tc,
        out_shape=jax.ShapeDtypeStruct((1, 1), jnp.float32),
        in_specs=[pl.BlockSpec(memory_space=pl.ANY)],
        scratch_shapes=[
            pltpu.VMEM((ROWS, COLS), jnp.float32),
            pltpu.SemaphoreType.DMA((NCHUNK,)),
        ],
    )(x2d)
    return out[0, 0]


# speedup vs baseline: 1.0249x; 1.0249x over previous
import jax
import jax.numpy as jnp
import numpy as np
from jax.experimental import pallas as pl

ROWS = 128
COLS = 128

THRESHOLD = np.float32(5e-8)
SCALE = np.float32(1e-7)


def _spl_loss_tc(x_ref, out_ref):
    x = x_ref[...]
    keep = (x * SCALE) < THRESHOLD
    y = jnp.where(keep, x, np.float32(0.0))
    part = jnp.sum(y, axis=0, keepdims=True)
    out_ref[...] = jnp.sum(part, axis=1, keepdims=True)


def kernel(super_loss, index, v):
    del index, v
    x2d = super_loss.reshape(ROWS, COLS)
    out = pl.pallas_call(
        _spl_loss_tc,
        out_shape=jax.ShapeDtypeStruct((1, 1), jnp.float32),
    )(x2d)
    return out[0, 0]
